# baseline (speedup 1.0000x reference)
import jax
import jax.numpy as jnp
from jax.experimental import pallas as pl

_D = 64
_K = 8192
_BM = 256
_SEG = 2048
_BETA = 0.25


def _vq_body(x_ref, w_ref, rn_ref, cn_ref, idx_ref, qst_ref, ls_ref):
    x = x_ref[...]
    m = jax.lax.dot_general(x.astype(jnp.bfloat16), w_ref[...],
                            (((1,), (1,)), ((), ())),
                            preferred_element_type=jnp.float32)
    d2 = rn_ref[...] - 2.0 * m + cn_ref[...]
    dist = jnp.sqrt(jnp.maximum(d2, 0.0))
    lane = jax.lax.broadcasted_iota(jnp.int32, (x.shape[0], _K), 1)

    acc_v = None
    for c in range(_K // _SEG):
        dc = dist[:, c * _SEG:(c + 1) * _SEG]
        lc = lane[:, c * _SEG:(c + 1) * _SEG]
        mn_c = jnp.min(dc, axis=1, keepdims=True)
        idx_c = jnp.min(jnp.where(dc == mn_c, lc, _K), axis=1, keepdims=True)
        if acc_v is None:
            acc_v = mn_c.astype(jnp.bfloat16).astype(jnp.float32)
            acc_i = idx_c
        else:
            upd = mn_c < acc_v
            acc_v = jnp.where(upd, mn_c.astype(jnp.bfloat16).astype(jnp.float32),
                              acc_v)
            acc_i = jnp.where(upd, idx_c, acc_i)
    idx_ref[...] = acc_i

    oh = (lane == acc_i).astype(jnp.bfloat16)
    q = jax.lax.dot_general(oh, w_ref[...].astype(jnp.bfloat16),
                            (((1,), (0,)), ((), ())),
                            preferred_element_type=jnp.float32)
    diff = q - x
    qst_ref[...] = x + diff
    part = jnp.sum(diff * diff).reshape(1, 1)

    @pl.when(pl.program_id(0) == 0)
    def _():
        ls_ref[...] = part

    @pl.when(pl.program_id(0) != 0)
    def _():
        ls_ref[...] += part


def kernel(encoding, W):
    shape = encoding.shape
    flat = encoding.reshape(-1, _D)
    n = flat.shape[0]
    rn = jnp.sum(flat * flat, axis=1, keepdims=True)
    cn = jnp.sum(W * W, axis=1)[None, :]
    idx, qst, ls = pl.pallas_call(
        _vq_body,
        grid=(n // _BM,),
        in_specs=[
            pl.BlockSpec((_BM, _D), lambda i: (i, 0)),
            pl.BlockSpec((_K, _D), lambda i: (0, 0)),
            pl.BlockSpec((_BM, 1), lambda i: (i, 0)),
            pl.BlockSpec((1, _K), lambda i: (0, 0)),
        ],
        out_specs=[
            pl.BlockSpec((_BM, 1), lambda i: (i, 0)),
            pl.BlockSpec((_BM, _D), lambda i: (i, 0)),
            pl.BlockSpec((1, 1), lambda i: (0, 0)),
        ],
        out_shape=[
            jax.ShapeDtypeStruct((n, 1), jnp.int32),
            jax.ShapeDtypeStruct((n, _D), jnp.float32),
            jax.ShapeDtypeStruct((1, 1), jnp.float32),
        ],
    )(flat, W, rn, cn)
    mean_sq = ls[0, 0] / flat.size
    vq_loss = mean_sq * _BETA + mean_sq
    return idx, qst.reshape(shape), vq_loss
